# baseline (speedup 1.0000x reference)
import jax
import jax.numpy as jnp
from jax.experimental import pallas as pl
from jax.experimental.pallas import tpu as pltpu

_B, _C, _H, _W = 16, 64, 32, 32
_HW = _H * _W
_K = 1024
_LOSS_SCALE = 1.25 / (_B * _C * _HW)
_BPS = 8


def _vq_body(z_ref, emb_ref, out_ref, loss_ref):
    emb = emb_ref[...]
    esq = jnp.sum(emb * emb, axis=1)

    part = jnp.float32(0.0)
    for j in range(_BPS):
        zb = z_ref[j]
        flat = zb.T

        mm = jax.lax.dot_general(
            flat, emb, (((1,), (1,)), ((), ())))
        s1 = jnp.sum(flat * flat, axis=1, keepdims=True)
        d = (s1 - 2.0 * mm) + esq[None, :]

        dmin = jnp.min(d, axis=1, keepdims=True)
        iota_k = jax.lax.broadcasted_iota(jnp.int32, (_HW, _K), 1)
        idx = jnp.min(jnp.where(d == dmin, iota_k, _K),
                      axis=1, keepdims=True)

        onehot = (iota_k == idx).astype(jnp.float32)
        q = jax.lax.dot_general(
            emb, onehot, (((0,), (1,)), ((), ())))

        diff = q - zb
        out_ref[j] = zb + diff
        part = part + jnp.sum(diff * diff)

    loss_ref[0, 0, 0] = part * _LOSS_SCALE


@jax.jit
def _vq(z3, embeddings):
    out, loss = pl.pallas_call(
        _vq_body,
        grid=(_B // _BPS,),
        in_specs=[
            pl.BlockSpec((_BPS, _C, _HW), lambda b: (b, 0, 0)),
            pl.BlockSpec((_K, _C), lambda b: (0, 0)),
        ],
        out_specs=[
            pl.BlockSpec((_BPS, _C, _HW), lambda b: (b, 0, 0)),
            pl.BlockSpec((1, 1, 1), lambda b: (b, 0, 0), memory_space=pltpu.SMEM),
        ],
        out_shape=[
            jax.ShapeDtypeStruct((_B, _C, _HW), jnp.float32),
            jax.ShapeDtypeStruct((_B // _BPS, 1, 1), jnp.float32),
        ],
        compiler_params=pltpu.CompilerParams(
            dimension_semantics=("parallel",)),
    )(z3, embeddings)
    return out, jnp.sum(loss)


def kernel(z, embeddings):
    z3 = z.reshape(_B, _C, _HW)
    out, loss = _vq(z3, embeddings)
    return out.reshape(_B, _C, _H, _W), loss


# speedup vs baseline: 1.1490x; 1.1490x over previous
import jax
import jax.numpy as jnp
from jax.experimental import pallas as pl
from jax.experimental.pallas import tpu as pltpu

_B, _C, _H, _W = 16, 64, 32, 32
_HW = _H * _W
_K = 1024
_LOSS_SCALE = 1.25 / (_B * _C * _HW)
_BPS = 4


def _vq_body(z_ref, emb_ref, out_ref, loss_ref):
    emb = emb_ref[...]
    esq = jnp.sum(emb * emb, axis=1)

    part = jnp.float32(0.0)
    for j in range(_BPS):
        zb = z_ref[j]

        mm = jax.lax.dot_general(
            emb, zb, (((1,), (0,)), ((), ())))
        s1 = jnp.sum(zb * zb, axis=0, keepdims=True)
        d = (s1 - 2.0 * mm) + esq[:, None]

        dmin = jnp.min(d, axis=0, keepdims=True)
        iota_k = jax.lax.broadcasted_iota(jnp.int32, (_K, _HW), 0)
        idx = jnp.min(jnp.where(d == dmin, iota_k, _K),
                      axis=0, keepdims=True)

        onehot = (iota_k == idx).astype(jnp.float32)
        q = jax.lax.dot_general(
            emb, onehot, (((0,), (0,)), ((), ())))

        diff = q - zb
        out_ref[j] = zb + diff
        part = part + jnp.sum(diff * diff)

    loss_ref[0, 0, 0] = part * _LOSS_SCALE


@jax.jit
def _vq(z3, embeddings):
    out, loss = pl.pallas_call(
        _vq_body,
        grid=(_B // _BPS,),
        in_specs=[
            pl.BlockSpec((_BPS, _C, _HW), lambda b: (b, 0, 0)),
            pl.BlockSpec((_K, _C), lambda b: (0, 0)),
        ],
        out_specs=[
            pl.BlockSpec((_BPS, _C, _HW), lambda b: (b, 0, 0)),
            pl.BlockSpec((1, 1, 1), lambda b: (b, 0, 0), memory_space=pltpu.SMEM),
        ],
        out_shape=[
            jax.ShapeDtypeStruct((_B, _C, _HW), jnp.float32),
            jax.ShapeDtypeStruct((_B // _BPS, 1, 1), jnp.float32),
        ],
        compiler_params=pltpu.CompilerParams(
            dimension_semantics=("parallel",)),
    )(z3, embeddings)
    return out, jnp.sum(loss)


def kernel(z, embeddings):
    z3 = z.reshape(_B, _C, _HW)
    out, loss = _vq(z3, embeddings)
    return out.reshape(_B, _C, _H, _W), loss


# speedup vs baseline: 1.1627x; 1.0119x over previous
import jax
import jax.numpy as jnp
from jax.experimental import pallas as pl
from jax.experimental.pallas import tpu as pltpu

_B, _C, _H, _W = 16, 64, 32, 32
_HW = _H * _W
_K = 1024
_LOSS_SCALE = 1.25 / (_B * _C * _HW)
_BPS = 4


def _vq_body(z_ref, emb_ref, out_ref, loss_ref):
    emb = emb_ref[...]
    esq = jnp.sum(emb * emb, axis=1)
    esqc = esq[:, None]
    emb2 = emb + emb

    part = jnp.float32(0.0)
    for j in range(_BPS):
        zb = z_ref[j]

        mm2 = jax.lax.dot_general(
            emb2, zb, (((1,), (0,)), ((), ())))
        s1 = jnp.sum(zb * zb, axis=0, keepdims=True)
        d = (s1 - mm2) + esqc

        dmin = jnp.min(d, axis=0, keepdims=True)
        iota_k = jax.lax.broadcasted_iota(jnp.int32, (_K, _HW), 0)
        idx = jnp.min(jnp.where(d == dmin, iota_k, _K),
                      axis=0, keepdims=True)

        onehot = (iota_k == idx).astype(jnp.float32)
        q = jax.lax.dot_general(
            emb, onehot, (((0,), (0,)), ((), ())),
            preferred_element_type=jnp.float32)

        diff = q - zb
        out_ref[j] = zb + diff
        part = part + jnp.sum(diff * diff)

    loss_ref[0, 0, 0] = part * _LOSS_SCALE


@jax.jit
def _vq(z3, embeddings):
    out, loss = pl.pallas_call(
        _vq_body,
        grid=(_B // _BPS,),
        in_specs=[
            pl.BlockSpec((_BPS, _C, _HW), lambda b: (b, 0, 0)),
            pl.BlockSpec((_K, _C), lambda b: (0, 0)),
        ],
        out_specs=[
            pl.BlockSpec((_BPS, _C, _HW), lambda b: (b, 0, 0)),
            pl.BlockSpec((1, 1, 1), lambda b: (b, 0, 0), memory_space=pltpu.SMEM),
        ],
        out_shape=[
            jax.ShapeDtypeStruct((_B, _C, _HW), jnp.float32),
            jax.ShapeDtypeStruct((_B // _BPS, 1, 1), jnp.float32),
        ],
        compiler_params=pltpu.CompilerParams(
            dimension_semantics=("parallel",)),
    )(z3, embeddings)
    return out, jnp.sum(loss)


def kernel(z, embeddings):
    z3 = z.reshape(_B, _C, _HW)
    out, loss = _vq(z3, embeddings)
    return out.reshape(_B, _C, _H, _W), loss
